# Initial kernel scaffold; baseline (speedup 1.0000x reference)
import jax
import jax.numpy as jnp
from jax.experimental import pallas as pl


def kernel(output1, output2, output3, anchor1, anchor2, anchor3, offset1, offset2, offset3, stride1, stride2, stride3):
    raise NotImplementedError("write your pallas kernel here")


# speedup vs baseline: 6.8924x; 6.8924x over previous
import functools

import jax
import jax.numpy as jnp
from jax.experimental import pallas as pl
from jax.experimental.pallas import tpu as pltpu

C = 3
THRESH = 0.05
IOU_T = 0.5
TOPK = 500
NEG = -1e30
K = 512
NI = 24


def _nms_body(sj_ref, st_ref, bj_ref, bt_ref, kept_ref, m2_ref):
    sj = sj_ref[...]
    validj = (sj > 0.0).astype(jnp.float32)
    x1j = bj_ref[0]
    y1j = bj_ref[1]
    x2j = bj_ref[2]
    y2j = bj_ref[3]
    areaj = jnp.maximum(x2j - x1j, 0.0) * jnp.maximum(y2j - y1j, 0.0)
    jiota = jax.lax.broadcasted_iota(jnp.int32, (NI, K), 1)

    CI = 64
    for ci in range(K // CI):
        sl = pl.ds(ci * CI, CI)
        x1i = bt_ref[0, sl, :]
        y1i = bt_ref[1, sl, :]
        x2i = bt_ref[2, sl, :]
        y2i = bt_ref[3, sl, :]
        validi = (st_ref[sl, :] > 0.0).astype(jnp.float32)
        areai = jnp.maximum(x2i - x1i, 0.0) * jnp.maximum(y2i - y1i, 0.0)
        xx1 = jnp.maximum(x1i[:, :, None], x1j[None, :, :])
        yy1 = jnp.maximum(y1i[:, :, None], y1j[None, :, :])
        xx2 = jnp.minimum(x2i[:, :, None], x2j[None, :, :])
        yy2 = jnp.minimum(y2i[:, :, None], y2j[None, :, :])
        inter = jnp.maximum(xx2 - xx1, 0.0) * jnp.maximum(yy2 - yy1, 0.0)
        union = areai[:, :, None] + areaj[None, :, :] - inter
        sup = ((inter / jnp.maximum(union, 1e-12)) > IOU_T).astype(jnp.float32)
        ii = ci * CI + jax.lax.broadcasted_iota(jnp.int32, (CI, NI, K), 0)
        jgt = (jax.lax.broadcasted_iota(jnp.int32, (CI, NI, K), 2) > ii)
        m2_ref[sl, :, :] = sup * validi[:, :, None] * jgt.astype(jnp.float32)

    def step(i, keep):
        row = m2_ref[i]
        ki = jnp.sum(jnp.where(jiota == i, keep, 0.0), axis=1, keepdims=True)
        return keep * (1.0 - row * ki)

    keep = jax.lax.fori_loop(0, TOPK, step, jnp.ones((NI, K), jnp.float32))
    kept_ref[...] = keep * validj


def _run_nms(vals, boxes_sel):
    sj = vals
    st = vals.T
    bj = jnp.transpose(boxes_sel, (2, 0, 1))
    bt = jnp.transpose(boxes_sel, (2, 1, 0))
    kept = pl.pallas_call(
        _nms_body,
        out_shape=jax.ShapeDtypeStruct((NI, K), jnp.float32),
        scratch_shapes=[pltpu.VMEM((K, NI, K), jnp.float32)],
    )(sj, st, bj, bt)
    return kept


def _decode(out, anchor, offset, stride):
    xy = jax.nn.sigmoid(out[..., 0:2])
    wh = jnp.exp(out[..., 2:4]) * anchor
    obj = jax.nn.sigmoid(out[..., 4:5])
    cls = jax.nn.sigmoid(out[..., 5:])
    ctr = (xy + offset) * stride
    half = wh * 0.5
    bbox = jnp.concatenate([ctr - half, ctr + half], axis=-1)
    scores = obj * cls
    return bbox, scores


def kernel(output1, output2, output3, anchor1, anchor2, anchor3,
           offset1, offset2, offset3, stride1, stride2, stride3):
    b1, s1 = _decode(output1, anchor1, offset1, jnp.asarray(stride1, output1.dtype))
    b2, s2 = _decode(output2, anchor2, offset2, jnp.asarray(stride2, output2.dtype))
    b3, s3 = _decode(output3, anchor3, offset3, jnp.asarray(stride3, output3.dtype))
    boxes = jnp.concatenate([b1, b2, b3], axis=1)
    scores = jnp.concatenate([s1, s2, s3], axis=1)
    B, N, _ = boxes.shape

    cls_scores = jnp.where(scores > THRESH, scores, NEG)
    cls_scores = jnp.transpose(cls_scores, (0, 2, 1))
    vals, idxs = jax.lax.top_k(cls_scores.reshape(NI, N), TOPK)
    vals = jnp.concatenate([vals, jnp.full((NI, K - TOPK), NEG, vals.dtype)], axis=1)
    idxs = jnp.concatenate([idxs, jnp.zeros((NI, K - TOPK), idxs.dtype)], axis=1)

    boxes_bc = jnp.broadcast_to(boxes[:, None], (B, C, N, 4)).reshape(NI, N, 4)
    boxes_sel = jnp.take_along_axis(boxes_bc, idxs[:, :, None], axis=1)

    kept = _run_nms(vals, boxes_sel) > 0.5

    surv = jnp.zeros((NI, N), bool)
    row = jnp.broadcast_to(jnp.arange(NI)[:, None], (NI, TOPK))
    surv = surv.at[row, idxs[:, :TOPK]].set(kept[:, :TOPK])
    surv = jnp.transpose(surv.reshape(B, C, N), (0, 2, 1))

    cid = jnp.arange(C, dtype=scores.dtype)
    out_ids = jnp.where(surv, cid, -1.0).reshape(B, N * C, 1)
    out_scores = jnp.where(surv, scores, -1.0).reshape(B, N * C, 1)
    out_boxes = jnp.broadcast_to(boxes[:, :, None, :], (B, N, C, 4)).reshape(B, N * C, 4)
    return out_ids, out_scores, out_boxes
